# baseline (speedup 1.0000x reference)
import functools

import jax
import jax.numpy as jnp
from jax import lax
from jax.experimental import pallas as pl
from jax.experimental.pallas import tpu as pltpu
from jax.experimental.pallas import tpu_sc as plsc

_BATCH = 4
_ROWS = 8192
_DIM = 256
_NC = 2
_NS = 16
_CHUNK = _ROWS // (_NC * _NS)
_NSUB = 4
_SUB = _CHUNK // _NSUB


def _sc_body(w_hbm, out_hbm, buf, in_sems, out_sems):
    wid = lax.axis_index("s") * _NC + lax.axis_index("c")
    base = wid * _CHUNK
    in_copies = [
        pltpu.make_async_copy(
            w_hbm.at[pl.ds(base + j * _SUB, _SUB)],
            buf.at[j],
            in_sems.at[j],
        )
        for j in range(_NSUB)
    ]
    for c in in_copies:
        c.start()
    out_copies = []
    for j in range(_NSUB):
        in_copies[j].wait()
        for b in range(_BATCH):
            c = pltpu.make_async_copy(
                buf.at[j],
                out_hbm.at[pl.ds(b * _ROWS + base + j * _SUB, _SUB)],
                out_sems.at[j, b],
            )
            c.start()
            out_copies.append(c)
    for c in out_copies:
        c.wait()


def kernel(tokens, W):
    del tokens
    mesh = plsc.VectorSubcoreMesh(core_axis_name="c", subcore_axis_name="s")
    run = functools.partial(
        pl.kernel,
        mesh=mesh,
        out_type=jax.ShapeDtypeStruct((_BATCH * _ROWS, _DIM), jnp.float32),
        scratch_types=[
            pltpu.VMEM((_NSUB, _SUB, _DIM), jnp.float32),
            pltpu.SemaphoreType.DMA((_NSUB,)),
            pltpu.SemaphoreType.DMA((_NSUB, _BATCH)),
        ],
    )(_sc_body)
    out2d = run(W)
    return out2d.reshape(_BATCH, _ROWS, _DIM)


# speedup vs baseline: 2.4855x; 2.4855x over previous
import jax
import jax.numpy as jnp
from jax.experimental import pallas as pl
from jax.experimental.pallas import tpu as pltpu

_BATCH = 4
_ROWS = 8192
_DIM = 256
_CHUNKS = (128, 256, 512, 1024, 2048, 2048, 1024, 512, 256, 192, 128, 64)
_OFFS = tuple(sum(_CHUNKS[:i]) for i in range(len(_CHUNKS)))
assert sum(_CHUNKS) == _ROWS


def _fanout_body(w_hbm, out_hbm, buf, in_sems, out_sems):
    in_copies = [
        pltpu.make_async_copy(
            w_hbm.at[pl.ds(off, n), :],
            buf.at[pl.ds(off, n), :],
            in_sems.at[i],
        )
        for i, (off, n) in enumerate(zip(_OFFS, _CHUNKS))
    ]
    for c in in_copies:
        c.start()
    out_copies = []
    for i, (off, n) in enumerate(zip(_OFFS, _CHUNKS)):
        in_copies[i].wait()
        for b in range(_BATCH):
            c = pltpu.make_async_copy(
                buf.at[pl.ds(off, n), :],
                out_hbm.at[b, pl.ds(off, n), :],
                out_sems.at[i, b],
            )
            c.start()
            out_copies.append(c)
    for c in out_copies:
        c.wait()


def kernel(tokens, W):
    del tokens
    return pl.pallas_call(
        _fanout_body,
        in_specs=[pl.BlockSpec(memory_space=pl.ANY)],
        out_specs=pl.BlockSpec(memory_space=pl.ANY),
        out_shape=jax.ShapeDtypeStruct((_BATCH, _ROWS, _DIM), jnp.float32),
        scratch_shapes=[
            pltpu.VMEM((_ROWS, _DIM), jnp.float32),
            pltpu.SemaphoreType.DMA((len(_CHUNKS),)),
            pltpu.SemaphoreType.DMA((len(_CHUNKS), _BATCH)),
        ],
    )(W)


# speedup vs baseline: 2.4897x; 1.0017x over previous
import jax
import jax.numpy as jnp
from jax.experimental import pallas as pl
from jax.experimental.pallas import tpu as pltpu

_BATCH = 4
_ROWS = 8192
_DIM = 256
_CHUNKS = (128, 256, 512, 1024, 2048, 2048, 1024, 512, 256, 192, 128, 64)
_OFFS = tuple(sum(_CHUNKS[:i]) for i in range(len(_CHUNKS)))
assert sum(_CHUNKS) == _ROWS


def _fanout_body(w_hbm, out_hbm, buf, in_sems, out_sems):
    in_copies = [
        pltpu.make_async_copy(
            w_hbm.at[pl.ds(off, n), :],
            buf.at[pl.ds(off, n), :],
            in_sems.at[i],
        )
        for i, (off, n) in enumerate(zip(_OFFS, _CHUNKS))
    ]
    for c in in_copies:
        c.start()
    out_copies = []
    for i, (off, n) in enumerate(zip(_OFFS, _CHUNKS)):
        in_copies[i].wait()
        for k in range(_BATCH):
            b = (k + i) % _BATCH
            c = pltpu.make_async_copy(
                buf.at[pl.ds(off, n), :],
                out_hbm.at[b, pl.ds(off, n), :],
                out_sems.at[i, b],
            )
            c.start()
            out_copies.append(c)
    for c in out_copies:
        c.wait()


def kernel(tokens, W):
    del tokens
    return pl.pallas_call(
        _fanout_body,
        in_specs=[pl.BlockSpec(memory_space=pl.ANY)],
        out_specs=pl.BlockSpec(memory_space=pl.ANY),
        out_shape=jax.ShapeDtypeStruct((_BATCH, _ROWS, _DIM), jnp.float32),
        scratch_shapes=[
            pltpu.VMEM((_ROWS, _DIM), jnp.float32),
            pltpu.SemaphoreType.DMA((len(_CHUNKS),)),
            pltpu.SemaphoreType.DMA((len(_CHUNKS), _BATCH)),
        ],
    )(W)
